# baseline (speedup 1.0000x reference)
import functools

import jax
import jax.numpy as jnp
from jax import lax
from jax.experimental import pallas as pl
from jax.experimental.pallas import tpu as pltpu
from jax.experimental.pallas import tpu_sc as plsc

N = 100000
E = 6400000

NC = 2
NS = 16
NW = NC * NS

FF = 8
CH = 3584
G0 = 42
G1 = 70
EP = NS * (G0 + G1) * CH

NT = 100096
RPS = NT // NS

BN = 6256
TCG = NT // BN

_mesh = plsc.VectorSubcoreMesh(
    core_axis_name="c", subcore_axis_name="s", num_cores=NC, num_subcores=NS)
_cparams = pltpu.CompilerParams(use_tc_tiling_on_sc=False)


def _tile_chunks(c, s):
  base = jnp.where(c == 0, s * G0, NS * G0 + s * G1)
  gp = jnp.where(c == 0, G0 // 2, G1 // 2)
  return base, gp


@functools.partial(
    pl.kernel,
    out_type=jax.ShapeDtypeStruct((NC, NT, FF), jnp.float32),
    mesh=_mesh,
    scratch_types=[
        [pltpu.VMEM((CH,), jnp.int32) for _ in range(2)],
        pltpu.VMEM((CH, FF), jnp.float32),
        pltpu.VMEM_SHARED((NT, FF), jnp.float32),
        pltpu.SemaphoreType.DMA,
        pltpu.SemaphoreType.DMA,
    ],
    compiler_params=_cparams,
)
def _deg_kernel(dst_hbm, zeros_hbm, ones_hbm, out_hbm, dstv, onesv, degs,
                isem, ssem):
  c = lax.axis_index("c")
  s = lax.axis_index("s")
  r0 = s * RPS
  pltpu.sync_copy(zeros_hbm.at[pl.ds(r0, RPS)], degs.at[pl.ds(r0, RPS)])
  pltpu.sync_copy(ones_hbm, onesv)
  plsc.subcore_barrier()
  base, gp = _tile_chunks(c, s)

  def idx_ref(g):
    return dst_hbm.at[pl.ds((base + g) * CH, CH)]

  pltpu.async_copy(idx_ref(0), dstv[0], isem)

  def body(gg, carry):
    a = 2 * gg

    @pl.when(gg == 0)
    def _():
      pltpu.make_async_copy(idx_ref(a), dstv[0], isem).wait()
      pltpu.async_copy(onesv, degs.at[dstv[0]], ssem, add=True)

    @pl.when(gg > 0)
    def _():
      pltpu.make_async_copy(idx_ref(a), dstv[0], isem).wait()
      pltpu.make_async_copy(onesv, degs.at[dstv[1]], ssem).wait()
      pltpu.async_copy(onesv, degs.at[dstv[0]], ssem, add=True)

    pltpu.async_copy(idx_ref(a + 1), dstv[1], isem)
    pltpu.make_async_copy(idx_ref(a + 1), dstv[1], isem).wait()
    pltpu.make_async_copy(onesv, degs.at[dstv[0]], ssem).wait()
    pltpu.async_copy(onesv, degs.at[dstv[1]], ssem, add=True)

    @pl.when(gg < gp - 1)
    def _():
      pltpu.async_copy(idx_ref(a + 2), dstv[0], isem)

    return carry

  lax.fori_loop(0, gp, body, 0)
  pltpu.make_async_copy(onesv, degs.at[dstv[1]], ssem).wait()
  plsc.subcore_barrier()
  pltpu.sync_copy(degs.at[pl.ds(r0, RPS)], out_hbm.at[c, pl.ds(r0, RPS)])


@functools.partial(
    pl.kernel,
    out_type=jax.ShapeDtypeStruct((NC, NT, FF), jnp.float32),
    mesh=_mesh,
    scratch_types=[
        [pltpu.VMEM((CH,), jnp.int32) for _ in range(2)],
        [pltpu.VMEM((CH,), jnp.int32) for _ in range(2)],
        [pltpu.VMEM((CH, FF), jnp.float32) for _ in range(2)],
        pltpu.VMEM_SHARED((NT, FF), jnp.float32),
        pltpu.SemaphoreType.DMA,
        pltpu.SemaphoreType.DMA,
        pltpu.SemaphoreType.DMA,
    ],
    compiler_params=_cparams,
)
def _edge_kernel(y_hbm, src_hbm, dst_hbm, zeros_hbm, out_hbm,
                 srcv, dstv, rowsv, aggs, isem, gsem, ssem):
  c = lax.axis_index("c")
  s = lax.axis_index("s")
  r0 = s * RPS
  pltpu.sync_copy(zeros_hbm.at[pl.ds(r0, RPS)], aggs.at[pl.ds(r0, RPS)])
  plsc.subcore_barrier()
  base, gp = _tile_chunks(c, s)

  def src_ref(g):
    return src_hbm.at[pl.ds((base + g) * CH, CH)]

  def dst_ref(g):
    return dst_hbm.at[pl.ds((base + g) * CH, CH)]

  def start_idx(g, p):
    pltpu.async_copy(src_ref(g), srcv[p], isem)
    pltpu.async_copy(dst_ref(g), dstv[p], isem)

  def wait_idx(g, p):
    pltpu.make_async_copy(src_ref(g), srcv[p], isem).wait()
    pltpu.make_async_copy(dst_ref(g), dstv[p], isem).wait()

  def drain_scatter(p):
    pltpu.make_async_copy(rowsv[p], aggs.at[dstv[p]], ssem).wait()

  start_idx(0, 0)

  def body(gg, carry):
    a = 2 * gg

    wait_idx(a, 0)
    pltpu.async_copy(y_hbm.at[srcv[0]], rowsv[0], gsem).wait()

    @pl.when(gg > 0)
    def _():
      drain_scatter(1)
    pltpu.async_copy(rowsv[0], aggs.at[dstv[0]], ssem, add=True)
    start_idx(a + 1, 1)

    wait_idx(a + 1, 1)
    pltpu.async_copy(y_hbm.at[srcv[1]], rowsv[1], gsem).wait()
    drain_scatter(0)
    pltpu.async_copy(rowsv[1], aggs.at[dstv[1]], ssem, add=True)

    @pl.when(gg < gp - 1)
    def _():
      start_idx(a + 2, 0)

    return carry

  lax.fori_loop(0, gp, body, 0)
  drain_scatter(1)
  plsc.subcore_barrier()
  pltpu.sync_copy(aggs.at[pl.ds(r0, RPS)], out_hbm.at[c, pl.ds(r0, RPS)])


def _pre_body(dp, x, w1, dinv_o, y_o):
  cnt = dp[0, :, 0:1] + dp[1, :, 0:1]
  dinv = lax.rsqrt(cnt + 1.0)
  dinv_o[...] = dinv
  y_o[...] = jnp.dot(x[...], w1[...], preferred_element_type=jnp.float32) * dinv


def _tc_pre(dp, x, w1):
  return pl.pallas_call(
      _pre_body,
      grid=(TCG,),
      in_specs=[
          pl.BlockSpec((NC, BN, FF), lambda i: (0, i, 0)),
          pl.BlockSpec((BN, 2), lambda i: (i, 0)),
          pl.BlockSpec((2, FF), lambda i: (0, 0)),
      ],
      out_specs=[
          pl.BlockSpec((BN, 1), lambda i: (i, 0)),
          pl.BlockSpec((BN, FF), lambda i: (i, 0)),
      ],
      out_shape=[
          jax.ShapeDtypeStruct((NT, 1), jnp.float32),
          jax.ShapeDtypeStruct((NT, FF), jnp.float32),
      ],
  )(dp, x, w1)


def _layer_body(ap, y, dinv, w, b, y_o):
  agg = ap[0] + ap[1] + y[...]
  h = jnp.tanh(agg * dinv[...] + b[...])
  rid = lax.broadcasted_iota(jnp.int32, (BN, 1), 0) + pl.program_id(0) * BN
  mask = (rid < N).astype(jnp.float32)
  y_o[...] = (jnp.dot(h, w[...], preferred_element_type=jnp.float32)
              * dinv[...] * mask)


def _tc_layer(ap, y, dinv, w, b):
  return pl.pallas_call(
      _layer_body,
      grid=(TCG,),
      in_specs=[
          pl.BlockSpec((NC, BN, FF), lambda i: (0, i, 0)),
          pl.BlockSpec((BN, FF), lambda i: (i, 0)),
          pl.BlockSpec((BN, 1), lambda i: (i, 0)),
          pl.BlockSpec((FF, FF), lambda i: (0, 0)),
          pl.BlockSpec((1, FF), lambda i: (0, 0)),
      ],
      out_specs=pl.BlockSpec((BN, FF), lambda i: (i, 0)),
      out_shape=jax.ShapeDtypeStruct((NT, FF), jnp.float32),
  )(ap, y, dinv, w, b)


def _final_body(ap, y, dinv, b3, wr, br, o):
  @pl.when(pl.program_id(0) == 0)
  def _():
    o[...] = jnp.float32(N) * br[...]

  agg = ap[0] + ap[1] + y[...]
  h = jnp.tanh(agg * dinv[...] + b3[...])
  rid = lax.broadcasted_iota(jnp.int32, (BN, 1), 0) + pl.program_id(0) * BN
  mask = (rid < N).astype(jnp.float32)
  v = jnp.dot(h * mask, wr[...], preferred_element_type=jnp.float32)
  o[...] += jnp.sum(v)


def _tc_final(ap, y, dinv, b3, wr, br):
  return pl.pallas_call(
      _final_body,
      grid=(TCG,),
      in_specs=[
          pl.BlockSpec((NC, BN, FF), lambda i: (0, i, 0)),
          pl.BlockSpec((BN, FF), lambda i: (i, 0)),
          pl.BlockSpec((BN, 1), lambda i: (i, 0)),
          pl.BlockSpec((1, FF), lambda i: (0, 0)),
          pl.BlockSpec((FF, 1), lambda i: (0, 0)),
          pl.BlockSpec((1, 1), lambda i: (0, 0)),
      ],
      out_specs=pl.BlockSpec((1, 1), lambda i: (0, 0)),
      out_shape=jax.ShapeDtypeStruct((1, 1), jnp.float32),
  )(ap, y, dinv, b3, wr, br)


def _pad_mat(w, rows, cols):
  out = jnp.zeros((rows, cols), jnp.float32)
  return out.at[:w.shape[0], :w.shape[1]].set(w.astype(jnp.float32))


def kernel(x, edge_index, W1, b1, W2, b2, W3, b3, Wr, br):
  src = edge_index[0].astype(jnp.int32)
  dst = edge_index[1].astype(jnp.int32)
  pad = jnp.full((EP - E,), N, jnp.int32)
  srcp = jnp.concatenate([src, pad])
  dstp = jnp.concatenate([dst, pad])

  zt = jnp.zeros((NT, FF), jnp.float32)
  ones = jnp.ones((CH, FF), jnp.float32)
  xp = jnp.concatenate([x.astype(jnp.float32), jnp.zeros((NT - N, 2), jnp.float32)])

  w1p = _pad_mat(W1, 2, FF)
  w2p = _pad_mat(W2, FF, FF)
  w3p = _pad_mat(W3, FF, FF)
  wrp = _pad_mat(Wr, FF, 1)
  b1p = _pad_mat(b1.reshape(1, -1), 1, FF)
  b2p = _pad_mat(b2.reshape(1, -1), 1, FF)
  b3p = _pad_mat(b3.reshape(1, -1), 1, FF)
  brp = br.reshape(1, 1).astype(jnp.float32)

  degp = _deg_kernel(dstp, zt, ones)
  dinv, y1 = _tc_pre(degp, xp, w1p)

  a1 = _edge_kernel(y1, srcp, dstp, zt)
  y2 = _tc_layer(a1, y1, dinv, w2p, b1p)

  a2 = _edge_kernel(y2, srcp, dstp, zt)
  y3 = _tc_layer(a2, y2, dinv, w3p, b2p)

  a3 = _edge_kernel(y3, srcp, dstp, zt)
  out = _tc_final(a3, y3, dinv, b3p, wrp, brp)
  return out[0, 0]


# speedup vs baseline: 1.5274x; 1.5274x over previous
import functools

import jax
import jax.numpy as jnp
from jax import lax
from jax.experimental import pallas as pl
from jax.experimental.pallas import tpu as pltpu
from jax.experimental.pallas import tpu_sc as plsc

N = 100000
E = 6400000

NC = 2
NS = 16
NW = NC * NS

FF = 8
CH = 3584
G0 = 52
G1 = 60
EP = NS * (G0 + G1) * CH

NT = 100096
RPS = NT // NS

NTR = NT // 16
BN = 3128
TCG = NTR // BN
NPB = BN * 16

_mesh = plsc.VectorSubcoreMesh(
    core_axis_name="c", subcore_axis_name="s", num_cores=NC, num_subcores=NS)
_cparams = pltpu.CompilerParams(use_tc_tiling_on_sc=False)


def _tile_chunks(c, s):
  base = jnp.where(c == 0, s * G0, NS * G0 + s * G1)
  gp = jnp.where(c == 0, G0 // 2, G1 // 2)
  return base, gp


@functools.partial(
    pl.kernel,
    out_type=jax.ShapeDtypeStruct((NC, NT, FF), jnp.float32),
    mesh=_mesh,
    scratch_types=[
        [pltpu.VMEM((CH,), jnp.int32) for _ in range(2)],
        pltpu.VMEM((CH, FF), jnp.float32),
        pltpu.VMEM_SHARED((NT, FF), jnp.float32),
        pltpu.SemaphoreType.DMA,
        pltpu.SemaphoreType.DMA,
    ],
    compiler_params=_cparams,
)
def _deg_kernel(dst_hbm, zeros_hbm, ones_hbm, out_hbm, dstv, onesv, degs,
                isem, ssem):
  c = lax.axis_index("c")
  s = lax.axis_index("s")
  r0 = s * RPS
  pltpu.sync_copy(zeros_hbm.at[pl.ds(r0, RPS)], degs.at[pl.ds(r0, RPS)])
  pltpu.sync_copy(ones_hbm, onesv)
  plsc.subcore_barrier()
  base, gp = _tile_chunks(c, s)

  def idx_ref(g):
    return dst_hbm.at[pl.ds((base + g) * CH, CH)]

  pltpu.async_copy(idx_ref(0), dstv[0], isem)

  def body(gg, carry):
    a = 2 * gg

    @pl.when(gg == 0)
    def _():
      pltpu.make_async_copy(idx_ref(a), dstv[0], isem).wait()
      pltpu.async_copy(onesv, degs.at[dstv[0]], ssem, add=True)

    @pl.when(gg > 0)
    def _():
      pltpu.make_async_copy(idx_ref(a), dstv[0], isem).wait()
      pltpu.make_async_copy(onesv, degs.at[dstv[1]], ssem).wait()
      pltpu.async_copy(onesv, degs.at[dstv[0]], ssem, add=True)

    pltpu.async_copy(idx_ref(a + 1), dstv[1], isem)
    pltpu.make_async_copy(idx_ref(a + 1), dstv[1], isem).wait()
    pltpu.make_async_copy(onesv, degs.at[dstv[0]], ssem).wait()
    pltpu.async_copy(onesv, degs.at[dstv[1]], ssem, add=True)

    @pl.when(gg < gp - 1)
    def _():
      pltpu.async_copy(idx_ref(a + 2), dstv[0], isem)

    return carry

  lax.fori_loop(0, gp, body, 0)
  pltpu.make_async_copy(onesv, degs.at[dstv[1]], ssem).wait()
  plsc.subcore_barrier()
  pltpu.sync_copy(degs.at[pl.ds(r0, RPS)], out_hbm.at[c, pl.ds(r0, RPS)])


@functools.partial(
    pl.kernel,
    out_type=jax.ShapeDtypeStruct((NC, NT, FF), jnp.float32),
    mesh=_mesh,
    scratch_types=[
        [pltpu.VMEM((CH,), jnp.int32) for _ in range(2)],
        [pltpu.VMEM((CH,), jnp.int32) for _ in range(2)],
        [pltpu.VMEM((CH, FF), jnp.float32) for _ in range(2)],
        pltpu.VMEM_SHARED((NT, FF), jnp.float32),
        pltpu.SemaphoreType.DMA,
        pltpu.SemaphoreType.DMA,
        pltpu.SemaphoreType.DMA,
    ],
    compiler_params=_cparams,
)
def _edge_kernel(y_hbm, src_hbm, dst_hbm, zeros_hbm, out_hbm,
                 srcv, dstv, rowsv, aggs, isem, gsem, ssem):
  c = lax.axis_index("c")
  s = lax.axis_index("s")
  r0 = s * RPS
  pltpu.sync_copy(zeros_hbm.at[pl.ds(r0, RPS)], aggs.at[pl.ds(r0, RPS)])
  plsc.subcore_barrier()
  base, gp = _tile_chunks(c, s)

  def src_ref(g):
    return src_hbm.at[pl.ds((base + g) * CH, CH)]

  def dst_ref(g):
    return dst_hbm.at[pl.ds((base + g) * CH, CH)]

  def start_idx(g, p):
    pltpu.async_copy(src_ref(g), srcv[p], isem)
    pltpu.async_copy(dst_ref(g), dstv[p], isem)

  def wait_idx(g, p):
    pltpu.make_async_copy(src_ref(g), srcv[p], isem).wait()
    pltpu.make_async_copy(dst_ref(g), dstv[p], isem).wait()

  def drain_scatter(p):
    pltpu.make_async_copy(rowsv[p], aggs.at[dstv[p]], ssem).wait()

  start_idx(0, 0)

  def body(gg, carry):
    a = 2 * gg

    wait_idx(a, 0)
    pltpu.async_copy(y_hbm.at[srcv[0]], rowsv[0], gsem).wait()

    @pl.when(gg > 0)
    def _():
      drain_scatter(1)
    pltpu.async_copy(rowsv[0], aggs.at[dstv[0]], ssem, add=True)
    start_idx(a + 1, 1)

    wait_idx(a + 1, 1)
    pltpu.async_copy(y_hbm.at[srcv[1]], rowsv[1], gsem).wait()
    drain_scatter(0)
    pltpu.async_copy(rowsv[1], aggs.at[dstv[1]], ssem, add=True)

    @pl.when(gg < gp - 1)
    def _():
      start_idx(a + 2, 0)

    return carry

  lax.fori_loop(0, gp, body, 0)
  drain_scatter(1)
  plsc.subcore_barrier()
  pltpu.sync_copy(aggs.at[pl.ds(r0, RPS)], out_hbm.at[c, pl.ds(r0, RPS)])


def _node_mask(pid):
  row = lax.broadcasted_iota(jnp.int32, (BN, 128), 0) + pid * BN
  lane = lax.broadcasted_iota(jnp.int32, (BN, 128), 1)
  nid = row * 16 + lane // 8
  return (nid < N).astype(jnp.float32)


def _pre_body(dp, x, wb1, dinv_o, y_o):
  cnt = dp[0] + dp[1]
  dinv = lax.rsqrt(cnt + 1.0)
  dinv_o[...] = dinv
  y_o[...] = jnp.dot(x[...], wb1[...], preferred_element_type=jnp.float32) * dinv


def _tc_pre(dp, x, wb1):
  return pl.pallas_call(
      _pre_body,
      grid=(TCG,),
      in_specs=[
          pl.BlockSpec((NC, BN, 128), lambda i: (0, i, 0)),
          pl.BlockSpec((BN, 128), lambda i: (i, 0)),
          pl.BlockSpec((128, 128), lambda i: (0, 0)),
      ],
      out_specs=[
          pl.BlockSpec((BN, 128), lambda i: (i, 0)),
          pl.BlockSpec((BN, 128), lambda i: (i, 0)),
      ],
      out_shape=[
          jax.ShapeDtypeStruct((NTR, 128), jnp.float32),
          jax.ShapeDtypeStruct((NTR, 128), jnp.float32),
      ],
  )(dp, x, wb1)


def _layer_body(ap, y, dinv, wb, bl, y_o):
  agg = ap[0] + ap[1] + y[...]
  h = jnp.tanh(agg * dinv[...] + bl[...])
  y_o[...] = (jnp.dot(h, wb[...], preferred_element_type=jnp.float32)
              * dinv[...] * _node_mask(pl.program_id(0)))


def _tc_layer(ap, y, dinv, wb, bl):
  return pl.pallas_call(
      _layer_body,
      grid=(TCG,),
      in_specs=[
          pl.BlockSpec((NC, BN, 128), lambda i: (0, i, 0)),
          pl.BlockSpec((BN, 128), lambda i: (i, 0)),
          pl.BlockSpec((BN, 128), lambda i: (i, 0)),
          pl.BlockSpec((128, 128), lambda i: (0, 0)),
          pl.BlockSpec((1, 128), lambda i: (0, 0)),
      ],
      out_specs=pl.BlockSpec((BN, 128), lambda i: (i, 0)),
      out_shape=jax.ShapeDtypeStruct((NTR, 128), jnp.float32),
  )(ap, y, dinv, wb, bl)


def _final_body(ap, y, dinv, bl3, wrl, br, o):
  @pl.when(pl.program_id(0) == 0)
  def _():
    o[...] = jnp.float32(N) * br[...]

  agg = ap[0] + ap[1] + y[...]
  h = jnp.tanh(agg * dinv[...] + bl3[...])
  o[...] += jnp.sum(h * wrl[...] * _node_mask(pl.program_id(0)))


def _tc_final(ap, y, dinv, bl3, wrl, br):
  return pl.pallas_call(
      _final_body,
      grid=(TCG,),
      in_specs=[
          pl.BlockSpec((NC, BN, 128), lambda i: (0, i, 0)),
          pl.BlockSpec((BN, 128), lambda i: (i, 0)),
          pl.BlockSpec((BN, 128), lambda i: (i, 0)),
          pl.BlockSpec((1, 128), lambda i: (0, 0)),
          pl.BlockSpec((1, 128), lambda i: (0, 0)),
          pl.BlockSpec((1, 1), lambda i: (0, 0)),
      ],
      out_specs=pl.BlockSpec((1, 1), lambda i: (0, 0)),
      out_shape=jax.ShapeDtypeStruct((1, 1), jnp.float32),
  )(ap, y, dinv, bl3, wrl, br)


def _pad_mat(w, rows, cols):
  out = jnp.zeros((rows, cols), jnp.float32)
  return out.at[:w.shape[0], :w.shape[1]].set(w.astype(jnp.float32))


def kernel(x, edge_index, W1, b1, W2, b2, W3, b3, Wr, br):
  src = edge_index[0].astype(jnp.int32)
  dst = edge_index[1].astype(jnp.int32)
  pad = jnp.full((EP - E,), N, jnp.int32)
  srcp = jnp.concatenate([src, pad])
  dstp = jnp.concatenate([dst, pad])

  zt = jnp.zeros((NT, FF), jnp.float32)
  ones = jnp.ones((CH, FF), jnp.float32)
  xp8 = jnp.concatenate(
      [_pad_mat(x.astype(jnp.float32), N, FF),
       jnp.zeros((NT - N, FF), jnp.float32)]).reshape(NTR, 128)

  eye16 = jnp.eye(16, dtype=jnp.float32)
  wb1 = jnp.kron(eye16, _pad_mat(W1, FF, FF))
  wb2 = jnp.kron(eye16, _pad_mat(W2, FF, FF))
  wb3 = jnp.kron(eye16, _pad_mat(W3, FF, FF))
  bl1 = jnp.tile(_pad_mat(b1.reshape(1, -1), 1, FF), (1, 16))
  bl2 = jnp.tile(_pad_mat(b2.reshape(1, -1), 1, FF), (1, 16))
  bl3 = jnp.tile(_pad_mat(b3.reshape(1, -1), 1, FF), (1, 16))
  wrl = jnp.tile(_pad_mat(Wr.reshape(1, -1), 1, FF), (1, 16))
  brp = br.reshape(1, 1).astype(jnp.float32)

  degp = _deg_kernel(dstp, zt, ones)
  dinv, y1 = _tc_pre(degp.reshape(NC, NTR, 128), xp8, wb1)

  a1 = _edge_kernel(y1.reshape(NT, FF), srcp, dstp, zt)
  y2 = _tc_layer(a1.reshape(NC, NTR, 128), y1, dinv, wb2, bl1)

  a2 = _edge_kernel(y2.reshape(NT, FF), srcp, dstp, zt)
  y3 = _tc_layer(a2.reshape(NC, NTR, 128), y2, dinv, wb3, bl2)

  a3 = _edge_kernel(y3.reshape(NT, FF), srcp, dstp, zt)
  out = _tc_final(a3.reshape(NC, NTR, 128), y3, dinv, bl3, wrl, brp)
  return out[0, 0]
